# baseline (speedup 1.0000x reference)
import functools

import jax
import jax.numpy as jnp
from jax import lax
from jax.experimental import pallas as pl
from jax.experimental.pallas import tpu as pltpu
from jax.experimental.pallas import tpu_sc as plsc

N = 10000
E = 320000
C_IN = 128
HID = 256
C_OUT = 128

NC = 2
NS = 16
NW = NC * NS

NPW = 320
NPAD = NW * NPW
RPT_LEN = NPW + 24
RPT_PAD = (NW - 1) * NPW + RPT_LEN
CH = 64
FB = 16
EPAD = E + CH

GA = -2.0 * 0.7978845608028654
GB = GA * 0.044715

VB = HID // 16


def _mm_ps_body(x_ref, w_ref, b1_ref, p_ref, s_ref):
    ps = jnp.dot(x_ref[...], w_ref[...], preferred_element_type=jnp.float32)
    p_ref[...] = ps[:, :HID]
    s_ref[...] = ps[:, HID:] + b1_ref[...]


def _mm_out_body(g_ref, w2_ref, b2_ref, rhi_ref, rlo_ref, o_ref):
    cnt = (rhi_ref[0, 0, :] - rlo_ref[0, 0, :]).astype(jnp.float32)
    scale = 1.0 / jnp.maximum(cnt, 1.0)
    gs = g_ref[...] * scale[:, None]
    y = jnp.dot(gs, w2_ref[...], preferred_element_type=jnp.float32)
    o_ref[...] = y + b2_ref[...] * (cnt > 0.0).astype(jnp.float32)[:, None]


def _sc_segment_gelu(p_hbm, s_hbm, idx_hbm, rpt_hbm, g_hbm,
                     rpt_v, idx2_v, rows2_v, xidx_v, xrows_v, s_grp, flush_v,
                     semi0, semi1, semg0, semg1, sem2):
    c = lax.axis_index("c")
    s = lax.axis_index("s")
    wid = s * NC + c
    n0 = wid * NPW

    pltpu.async_copy(rpt_hbm.at[pl.ds(pl.multiple_of(n0, 8), RPT_LEN)],
                     rpt_v, sem2).wait()

    def rv(k):
        return rpt_v[pl.ds(k, 16)][0]

    zeros16 = jnp.zeros((16,), jnp.float32)

    def issue_idx(i, idx_ref, sem):
        a = pl.multiple_of((rv(i) // 8) * 8, 8)
        return pltpu.async_copy(idx_hbm.at[pl.ds(a, CH)], idx_ref, sem)

    def drain_idx(idx_ref, sem):
        pltpu.make_async_copy(idx_hbm.at[pl.ds(0, CH)], idx_ref, sem).wait()

    def drain_rows(rows_ref, sem):
        pltpu.make_async_copy(p_hbm.at[pl.ds(0, CH)], rows_ref, sem).wait()

    def make_edge_body(rows_ref, srow):
        def edge_body(r, acc_):
            new_acc = []
            for j in range(VB):
                x = rows_ref[r, pl.ds(j * 16, 16)] + srow[j]
                z = x * (GA + GB * (x * x))
                new_acc.append(acc_[j] + x / (1.0 + jnp.exp(z)))
            return tuple(new_acc)
        return edge_body

    def process(i, idx_this, rows_this, semg_this, semi_this,
                idx_other, rows_other, semg_other, semi_other,
                do_sload, do_flush):
        e0 = rv(i)
        e1 = rv(i + 1)

        if do_sload:
            @pl.when(i % FB == 0)
            def _():
                pltpu.async_copy(
                    s_hbm.at[pl.ds(pl.multiple_of(n0 + i, 8), FB)],
                    s_grp, sem2).wait()

        drain_rows(rows_this, semg_this)
        issue_idx(i + 2, idx_this, semi_this)
        drain_idx(idx_other, semi_other)
        pltpu.async_copy(p_hbm.at[idx_other], rows_other, semg_other)

        srow = tuple(s_grp[i % FB, pl.ds(j * 16, 16)] for j in range(VB))
        a0 = (e0 // 8) * 8
        lo = e0 - a0
        hi = jnp.minimum(e1, a0 + CH) - a0
        acc = lax.fori_loop(lo, hi, make_edge_body(rows_this, srow),
                            (zeros16,) * VB)

        nch = (e1 - a0 + CH - 1) // CH

        def xchunk(k, acc_):
            a = a0 + k * CH
            pltpu.async_copy(idx_hbm.at[pl.ds(pl.multiple_of(a, 8), CH)],
                             xidx_v, sem2).wait()
            pltpu.async_copy(p_hbm.at[xidx_v], xrows_v, sem2).wait()
            hi2 = jnp.minimum(e1, a + CH) - a
            return lax.fori_loop(0, hi2, make_edge_body(xrows_v, srow), acc_)

        acc = lax.fori_loop(1, nch, xchunk, acc)

        for j in range(VB):
            flush_v[i % FB, pl.ds(j * 16, 16)] = acc[j]

        if do_flush:
            @pl.when(i % FB == FB - 1)
            def _():
                pltpu.async_copy(
                    flush_v,
                    g_hbm.at[pl.ds(
                        pl.multiple_of(n0 + (i // FB) * FB, 8), FB)],
                    sem2).wait()

    issue_idx(0, idx2_v.at[0], semi0).wait()
    pltpu.async_copy(p_hbm.at[idx2_v.at[0]], rows2_v.at[0], semg0)
    issue_idx(1, idx2_v.at[1], semi1)

    def pair_body(t, _):
        i0 = 2 * t
        process(i0, idx2_v.at[0], rows2_v.at[0], semg0, semi0,
                idx2_v.at[1], rows2_v.at[1], semg1, semi1,
                do_sload=True, do_flush=False)
        process(i0 + 1, idx2_v.at[1], rows2_v.at[1], semg1, semi1,
                idx2_v.at[0], rows2_v.at[0], semg0, semi0,
                do_sload=False, do_flush=True)
        return 0

    lax.fori_loop(0, NPW // 2, pair_body, 0)

    drain_idx(idx2_v.at[1], semi1)
    drain_rows(rows2_v.at[0], semg0)


@functools.partial(
    pl.kernel,
    mesh=plsc.VectorSubcoreMesh(core_axis_name="c", subcore_axis_name="s"),
    out_type=jax.ShapeDtypeStruct((NPAD, HID), jnp.float32),
    scratch_types=[
        pltpu.VMEM((RPT_LEN,), jnp.int32),
        pltpu.VMEM((2, CH), jnp.int32),
        pltpu.VMEM((2, CH, HID), jnp.float32),
        pltpu.VMEM((CH,), jnp.int32),
        pltpu.VMEM((CH, HID), jnp.float32),
        pltpu.VMEM((FB, HID), jnp.float32),
        pltpu.VMEM((FB, HID), jnp.float32),
        pltpu.SemaphoreType.DMA,
        pltpu.SemaphoreType.DMA,
        pltpu.SemaphoreType.DMA,
        pltpu.SemaphoreType.DMA,
        pltpu.SemaphoreType.DMA,
    ],
)
def _sc_kernel(p_hbm, s_hbm, idx_hbm, rpt_hbm, g_hbm,
               rpt_v, idx2_v, rows2_v, xidx_v, xrows_v, s_grp, flush_v,
               semi0, semi1, semg0, semg1, sem2):
    _sc_segment_gelu(p_hbm, s_hbm, idx_hbm, rpt_hbm, g_hbm,
                     rpt_v, idx2_v, rows2_v, xidx_v, xrows_v, s_grp, flush_v,
                     semi0, semi1, semg0, semg1, sem2)


def kernel(in_features, W1, b1, W2, b2, neighbor_idx, rowptr):
    x = in_features[0]
    xp = jnp.pad(x, ((0, NPAD - N), (0, 0)))
    wc = jnp.concatenate([W1[:C_IN], W1[C_IN:]], axis=1)
    b1r = b1.reshape(1, HID)

    nblk = NPAD // 512
    p_arr, s_arr = pl.pallas_call(
        _mm_ps_body,
        grid=(nblk,),
        in_specs=[
            pl.BlockSpec((512, C_IN), lambda i: (i, 0)),
            pl.BlockSpec((C_IN, 2 * HID), lambda i: (0, 0)),
            pl.BlockSpec((1, HID), lambda i: (0, 0)),
        ],
        out_specs=[
            pl.BlockSpec((512, HID), lambda i: (i, 0)),
            pl.BlockSpec((512, HID), lambda i: (i, 0)),
        ],
        out_shape=[
            jax.ShapeDtypeStruct((NPAD, HID), jnp.float32),
            jax.ShapeDtypeStruct((NPAD, HID), jnp.float32),
        ],
    )(xp, wc, b1r)

    idx32 = neighbor_idx.astype(jnp.int32)
    rpt32 = rowptr.astype(jnp.int32)
    idxp = jnp.pad(idx32, (0, EPAD - E))
    rptp = jnp.pad(rpt32, (0, RPT_PAD - (N + 1)), constant_values=E)

    g_arr = _sc_kernel(p_arr, s_arr, idxp, rptp)

    rhi = rptp[1:NPAD + 1].reshape(nblk, 1, 512)
    rlo = rptp[:NPAD].reshape(nblk, 1, 512)
    b2r = b2.reshape(1, C_OUT)

    out = pl.pallas_call(
        _mm_out_body,
        grid=(nblk,),
        in_specs=[
            pl.BlockSpec((512, HID), lambda i: (i, 0)),
            pl.BlockSpec((HID, C_OUT), lambda i: (0, 0)),
            pl.BlockSpec((1, C_OUT), lambda i: (0, 0)),
            pl.BlockSpec((1, 1, 512), lambda i: (i, 0, 0)),
            pl.BlockSpec((1, 1, 512), lambda i: (i, 0, 0)),
        ],
        out_specs=pl.BlockSpec((512, C_OUT), lambda i: (i, 0)),
        out_shape=jax.ShapeDtypeStruct((NPAD, C_OUT), jnp.float32),
    )(g_arr, W2, b2r, rhi, rlo)

    return out[:N].reshape(1, N, C_OUT)


# speedup vs baseline: 1.0868x; 1.0868x over previous
import functools

import jax
import jax.numpy as jnp
from jax import lax
from jax.experimental import pallas as pl
from jax.experimental.pallas import tpu as pltpu
from jax.experimental.pallas import tpu_sc as plsc

N = 10000
E = 320000
C_IN = 128
HID = 256
C_OUT = 128

NC = 2
NS = 16
NW = NC * NS

NPW = 320
NPAD = NW * NPW
RPT_LEN = NPW + 24
RPT_PAD = (NW - 1) * NPW + RPT_LEN
CH = 32
FB = 16
EPAD = E + CH

GA = -2.0 * 0.7978845608028654
GB = GA * 0.044715

VB = HID // 16


def _mm_ps_body(x_ref, w_ref, b1_ref, p_ref, s_ref):
    ps = jnp.dot(x_ref[...], w_ref[...], preferred_element_type=jnp.float32)
    p_ref[...] = ps[:, :HID]
    s_ref[...] = ps[:, HID:] + b1_ref[...]


def _mm_out_body(g_ref, w2_ref, b2_ref, rhi_ref, rlo_ref, o_ref):
    cnt = (rhi_ref[0, 0, :] - rlo_ref[0, 0, :]).astype(jnp.float32)
    scale = 1.0 / jnp.maximum(cnt, 1.0)
    gs = g_ref[...] * scale[:, None]
    y = jnp.dot(gs, w2_ref[...], preferred_element_type=jnp.float32)
    o_ref[...] = y + b2_ref[...] * (cnt > 0.0).astype(jnp.float32)[:, None]


def _sc_segment_gelu(p_hbm, s_hbm, idx_hbm, rpt_hbm, g_hbm,
                     rpt_v, idx_v, rows_v, s_all, flush_v, sem, sem2):
    c = lax.axis_index("c")
    s = lax.axis_index("s")
    wid = s * NC + c
    n0 = wid * NPW

    pltpu.async_copy(rpt_hbm.at[pl.ds(pl.multiple_of(n0, 8), RPT_LEN)],
                     rpt_v, sem2).wait()
    pltpu.async_copy(s_hbm.at[pl.ds(pl.multiple_of(n0, 8), NPW)],
                     s_all, sem2).wait()

    def rv(k):
        return rpt_v[pl.ds(k, 16)][0]

    zeros16 = jnp.zeros((16,), jnp.float32)

    def node_body(i, _):
        e0 = rv(i)
        e1 = rv(i + 1)
        srow = tuple(s_all[i, pl.ds(j * 16, 16)] for j in range(VB))
        a0 = (e0 // 8) * 8
        nch = (e1 - a0 + CH - 1) // CH

        def chunk_body(k, acc):
            a = a0 + k * CH
            pltpu.async_copy(idx_hbm.at[pl.ds(pl.multiple_of(a, 8), CH)],
                             idx_v, sem2).wait()
            pltpu.async_copy(p_hbm.at[idx_v], rows_v, sem).wait()
            lo = jnp.maximum(e0, a) - a
            hi = jnp.minimum(e1, a + CH) - a

            def edge_body(r, acc_):
                new_acc = []
                for j in range(VB):
                    x = rows_v[r, pl.ds(j * 16, 16)] + srow[j]
                    z = x * (GA + GB * (x * x))
                    new_acc.append(acc_[j] + x / (1.0 + jnp.exp(z)))
                return tuple(new_acc)

            return lax.fori_loop(lo, hi, edge_body, acc)

        acc = lax.fori_loop(0, nch, chunk_body, (zeros16,) * VB)
        for j in range(VB):
            flush_v[i % FB, pl.ds(j * 16, 16)] = acc[j]

        @pl.when(i % FB == FB - 1)
        def _():
            pltpu.async_copy(
                flush_v,
                g_hbm.at[pl.ds(pl.multiple_of(n0 + (i // FB) * FB, 8), FB)],
                sem2).wait()

        return 0

    lax.fori_loop(0, NPW, node_body, 0)


@functools.partial(
    pl.kernel,
    mesh=plsc.VectorSubcoreMesh(core_axis_name="c", subcore_axis_name="s"),
    out_type=jax.ShapeDtypeStruct((NPAD, HID), jnp.float32),
    scratch_types=[
        pltpu.VMEM((RPT_LEN,), jnp.int32),
        pltpu.VMEM((CH,), jnp.int32),
        pltpu.VMEM((CH, HID), jnp.float32),
        pltpu.VMEM((NPW, HID), jnp.float32),
        pltpu.VMEM((FB, HID), jnp.float32),
        pltpu.SemaphoreType.DMA,
        pltpu.SemaphoreType.DMA,
    ],
)
def _sc_kernel(p_hbm, s_hbm, idx_hbm, rpt_hbm, g_hbm,
               rpt_v, idx_v, rows_v, s_all, flush_v, sem, sem2):
    _sc_segment_gelu(p_hbm, s_hbm, idx_hbm, rpt_hbm, g_hbm,
                     rpt_v, idx_v, rows_v, s_all, flush_v, sem, sem2)


def kernel(in_features, W1, b1, W2, b2, neighbor_idx, rowptr):
    x = in_features[0]
    xp = jnp.pad(x, ((0, NPAD - N), (0, 0)))
    wc = jnp.concatenate([W1[:C_IN], W1[C_IN:]], axis=1)
    b1r = b1.reshape(1, HID)

    nblk = NPAD // 512
    p_arr, s_arr = pl.pallas_call(
        _mm_ps_body,
        grid=(nblk,),
        in_specs=[
            pl.BlockSpec((512, C_IN), lambda i: (i, 0)),
            pl.BlockSpec((C_IN, 2 * HID), lambda i: (0, 0)),
            pl.BlockSpec((1, HID), lambda i: (0, 0)),
        ],
        out_specs=[
            pl.BlockSpec((512, HID), lambda i: (i, 0)),
            pl.BlockSpec((512, HID), lambda i: (i, 0)),
        ],
        out_shape=[
            jax.ShapeDtypeStruct((NPAD, HID), jnp.float32),
            jax.ShapeDtypeStruct((NPAD, HID), jnp.float32),
        ],
    )(xp, wc, b1r)

    idx32 = neighbor_idx.astype(jnp.int32)
    rpt32 = rowptr.astype(jnp.int32)
    idxp = jnp.pad(idx32, (0, EPAD - E))
    rptp = jnp.pad(rpt32, (0, RPT_PAD - (N + 1)), constant_values=E)

    g_arr = _sc_kernel(p_arr, s_arr, idxp, rptp)

    rhi = rptp[1:NPAD + 1].reshape(nblk, 1, 512)
    rlo = rptp[:NPAD].reshape(nblk, 1, 512)
    b2r = b2.reshape(1, C_OUT)

    out = pl.pallas_call(
        _mm_out_body,
        grid=(nblk,),
        in_specs=[
            pl.BlockSpec((512, HID), lambda i: (i, 0)),
            pl.BlockSpec((HID, C_OUT), lambda i: (0, 0)),
            pl.BlockSpec((1, C_OUT), lambda i: (0, 0)),
            pl.BlockSpec((1, 1, 512), lambda i: (i, 0, 0)),
            pl.BlockSpec((1, 1, 512), lambda i: (i, 0, 0)),
        ],
        out_specs=pl.BlockSpec((512, C_OUT), lambda i: (i, 0)),
        out_shape=jax.ShapeDtypeStruct((NPAD, C_OUT), jnp.float32),
    )(g_arr, W2, b2r, rhi, rlo)

    return out[:N].reshape(1, N, C_OUT)
